# baseline (speedup 1.0000x reference)
import jax
import jax.numpy as jnp
from jax.experimental import pallas as pl
from jax.experimental.pallas import tpu as pltpu

SEQ_BLOCK = 1024


def _scan_body(x_ref, o_ref, carry_ref):
    j = pl.program_id(1)

    @pl.when(j == 0)
    def _():
        carry_ref[...] = jnp.zeros_like(carry_ref)

    xb = x_ref[0]
    s = xb.shape[0]
    row = jax.lax.broadcasted_iota(jnp.int32, (s, s), 0)
    col = jax.lax.broadcasted_iota(jnp.int32, (s, s), 1)
    tri = (col < row).astype(jnp.float32)
    excl = jax.lax.dot(tri, xb, preferred_element_type=jnp.float32)
    carry = carry_ref[0]
    out = excl + carry[None, :]
    o_ref[0] = out
    carry_ref[0] = out[-1] + xb[-1]


@jax.jit
def kernel(x):
    b, n, c = x.shape
    grid = (b, n // SEQ_BLOCK)
    return pl.pallas_call(
        _scan_body,
        grid=grid,
        in_specs=[
            pl.BlockSpec((1, SEQ_BLOCK, c), lambda i, j: (i, j, 0)),
        ],
        out_specs=pl.BlockSpec((1, SEQ_BLOCK, c), lambda i, j: (i, j, 0)),
        out_shape=jax.ShapeDtypeStruct((b, n, c), x.dtype),
        scratch_shapes=[pltpu.VMEM((1, c), jnp.float32)],
        compiler_params=pltpu.CompilerParams(
            dimension_semantics=("arbitrary", "arbitrary"),
        ),
    )(x)


# speedup vs baseline: 1.1211x; 1.1211x over previous
import jax
import jax.numpy as jnp
from jax.experimental import pallas as pl
from jax.experimental.pallas import tpu as pltpu

SEQ_BLOCK = 1024
SUB = 256


def _scan_body(x_ref, o_ref, carry_ref):
    j = pl.program_id(1)

    @pl.when(j == 0)
    def _():
        carry_ref[...] = jnp.zeros_like(carry_ref)

    row = jax.lax.broadcasted_iota(jnp.int32, (SUB, SUB), 0)
    col = jax.lax.broadcasted_iota(jnp.int32, (SUB, SUB), 1)
    tri = (col < row).astype(jnp.float32)

    off = carry_ref[0]
    for g in range(SEQ_BLOCK // SUB):
        xg = x_ref[0, g * SUB:(g + 1) * SUB, :]
        excl = jax.lax.dot(tri, xg, preferred_element_type=jnp.float32)
        o_ref[0, g * SUB:(g + 1) * SUB, :] = excl + off[None, :]
        off = off + excl[-1] + xg[-1]
    carry_ref[0] = off


@jax.jit
def kernel(x):
    b, n, c = x.shape
    grid = (b, n // SEQ_BLOCK)
    return pl.pallas_call(
        _scan_body,
        grid=grid,
        in_specs=[
            pl.BlockSpec((1, SEQ_BLOCK, c), lambda i, j: (i, j, 0)),
        ],
        out_specs=pl.BlockSpec((1, SEQ_BLOCK, c), lambda i, j: (i, j, 0)),
        out_shape=jax.ShapeDtypeStruct((b, n, c), x.dtype),
        scratch_shapes=[pltpu.VMEM((1, c), jnp.float32)],
        compiler_params=pltpu.CompilerParams(
            dimension_semantics=("arbitrary", "arbitrary"),
        ),
    )(x)


# speedup vs baseline: 1.1240x; 1.0026x over previous
import jax
import jax.numpy as jnp
from jax.experimental import pallas as pl
from jax.experimental.pallas import tpu as pltpu

SEQ_BLOCK = 1024
SUB = 128


def _scan_body(x_ref, o_ref, carry_ref):
    j = pl.program_id(1)

    @pl.when(j == 0)
    def _():
        carry_ref[...] = jnp.zeros_like(carry_ref)

    row = jax.lax.broadcasted_iota(jnp.int32, (SUB, SUB), 0)
    col = jax.lax.broadcasted_iota(jnp.int32, (SUB, SUB), 1)
    tri = (col < row).astype(jnp.float32)

    off = carry_ref[0]
    for g in range(SEQ_BLOCK // SUB):
        xg = x_ref[0, g * SUB:(g + 1) * SUB, :]
        excl = jax.lax.dot(tri, xg, preferred_element_type=jnp.float32)
        o_ref[0, g * SUB:(g + 1) * SUB, :] = excl + off[None, :]
        off = off + excl[-1] + xg[-1]
    carry_ref[0] = off


@jax.jit
def kernel(x):
    b, n, c = x.shape
    grid = (b, n // SEQ_BLOCK)
    return pl.pallas_call(
        _scan_body,
        grid=grid,
        in_specs=[
            pl.BlockSpec((1, SEQ_BLOCK, c), lambda i, j: (i, j, 0)),
        ],
        out_specs=pl.BlockSpec((1, SEQ_BLOCK, c), lambda i, j: (i, j, 0)),
        out_shape=jax.ShapeDtypeStruct((b, n, c), x.dtype),
        scratch_shapes=[pltpu.VMEM((1, c), jnp.float32)],
        compiler_params=pltpu.CompilerParams(
            dimension_semantics=("arbitrary", "arbitrary"),
        ),
    )(x)
